# baseline (speedup 1.0000x reference)
import jax
import jax.numpy as jnp
from jax.experimental import pallas as pl
from jax.experimental.pallas import tpu as pltpu

_N = 100000
_C = 80
_BLK = 4000
_GRID = _N // _BLK


def _tc_body(pred_ref, lab_ref, score_ref, w_ref, out_ref):
    p = pred_ref[...]
    lab = lab_ref[...]
    s = score_ref[...]
    w = w_ref[...]
    log1mp = jnp.maximum(jnp.log(1.0 - p), -100.0)
    neg = log1mp * (p * p * -0.75)
    labc = jnp.clip(lab, 0, _C - 1)
    onehot = jax.lax.broadcasted_iota(jnp.int32, (_BLK, _C), 1) == labc
    pos_mask = (lab >= 0) & (lab < _C)
    p_at = jnp.sum(jnp.where(onehot, p, 0.0), axis=1, keepdims=True)
    neg_at = jnp.sum(jnp.where(onehot, neg, 0.0), axis=1, keepdims=True)
    lp = jnp.maximum(jnp.log(p_at), -100.0)
    l1p = jnp.maximum(jnp.log(1.0 - p_at), -100.0)
    pos_val = -(s * lp + (1.0 - s) * l1p) * w
    corr = jnp.where(pos_mask, pos_val - neg_at, 0.0)
    total = jnp.sum(neg) + jnp.sum(corr)

    @pl.when(pl.program_id(0) == 0)
    def _init():
        out_ref[0, 0] = 0.0

    out_ref[0, 0] += total


def kernel(pred, label, score, weight):
    lab2 = label.reshape(_N, 1)
    s2 = score.reshape(_N, 1)
    w2 = weight.reshape(_N, 1)
    out = pl.pallas_call(
        _tc_body,
        grid=(_GRID,),
        in_specs=[
            pl.BlockSpec((_BLK, _C), lambda i: (i, 0)),
            pl.BlockSpec((_BLK, 1), lambda i: (i, 0)),
            pl.BlockSpec((_BLK, 1), lambda i: (i, 0)),
            pl.BlockSpec((_BLK, 1), lambda i: (i, 0)),
        ],
        out_specs=pl.BlockSpec((1, 1), lambda i: (0, 0), memory_space=pltpu.SMEM),
        out_shape=jax.ShapeDtypeStruct((1, 1), jnp.float32),
    )(pred, lab2, s2, w2)
    return out[0, 0] / _N
